# baseline (speedup 1.0000x reference)
import jax
import jax.numpy as jnp
from jax import lax
from jax.experimental import pallas as pl
from jax.experimental.pallas import tpu as pltpu
from jax.experimental.pallas import tpu_sc as plsc

N = 50000
E = 300000
D = 128
L = 16
NS = 16
RSH = 12
R = 1 << RSH
NR = (N + R - 1) // R
ACC_ROWS = R + 128
ROWS_PT = ACC_ROWS // NS
DUMMY = R
B = 128
CG = 128
EG = E // L
GPT_LO = EG // NS
CLIST = 20736
NCHUNK = (GPT_LO + 1 + CG - 1) // CG


def _linear_body(x_ref, w_ref, b_ref, o_ref):
    o_ref[...] = (
        jnp.dot(x_ref[...], w_ref[0], preferred_element_type=jnp.float32)
        + b_ref[0]
    )


def _tc_linear(x_cat, wt_stack, b_stack):
    blk = 1000
    grid = (2 * N) // blk
    return pl.pallas_call(
        _linear_body,
        grid=(grid,),
        in_specs=[
            pl.BlockSpec((blk, D), lambda i: (i, 0)),
            pl.BlockSpec((1, D, D), lambda i: (i // (grid // 2), 0, 0)),
            pl.BlockSpec((1, 1, D), lambda i: (i // (grid // 2), 0, 0)),
        ],
        out_specs=pl.BlockSpec((blk, D), lambda i: (i, 0)),
        out_shape=jax.ShapeDtypeStruct((2 * N, D), jnp.float32),
    )(x_cat, wt_stack, b_stack)


def _sc_body(wh_hbm, src_hbm, dst_hbm, hb_out, ha_out,
             src_st, dst_st, clist, csb_v, dlb_v,
             rows_v, ones_v, zsrc_v, zcnt_v, fblk_v, cblk_v,
             hist_v, cur_v, poff_v, ramp_v,
             acc, cnts, gsem):
    c = lax.axis_index("c")
    s = lax.axis_index("s")
    i32 = jnp.int32

    zv = jnp.zeros((L,), jnp.float32)
    ov = jnp.ones((L,), jnp.float32)

    def _init(i, _):
        @pl.when(i < 32)
        def _():
            for j in range(D // L):
                zsrc_v[i, pl.ds(j * L, L)] = zv
        ones_v[i, pl.ds(0, L)] = ov
        zcnt_v[i, pl.ds(0, L)] = zv
        return 0

    lax.fori_loop(0, B, _init, 0)

    g0 = s * GPT_LO + jnp.minimum(s, 14)
    ng = jnp.where(s < 14, GPT_LO + 1, GPT_LO)
    ebase = c * E + g0 * L

    def _stage(ch):
        pltpu.sync_copy(src_hbm.at[pl.ds(ebase + ch * CG * L, CG * L)],
                        src_st)
        pltpu.sync_copy(dst_hbm.at[pl.ds(ebase + ch * CG * L, CG * L)],
                        dst_st)
        return jnp.minimum(CG, ng - ch * CG)

    hist_v[pl.ds(0, L)] = jnp.zeros((L,), i32)

    def _hchunk(ch, _):
        ngrp = _stage(ch)

        def _grp(gi, _):
            d16 = dst_st[pl.ds(gi * L, L)]
            rid = d16 >> RSH
            cnt, last = plsc.scan_count(rid)
            plsc.addupdate_scatter(hist_v, [rid], cnt, mask=last)
            return 0

        return lax.fori_loop(0, ngrp, _grp, 0)

    lax.fori_loop(0, NCHUNK, _hchunk, 0)

    histv = hist_v[pl.ds(0, L)]
    regionv = (((histv + (B - 1)) >> 7) << 7) + L
    poffv = plsc.cumsum(regionv) - regionv
    poff_v[pl.ds(0, L)] = poffv
    cur_v[pl.ds(0, L)] = poffv
    ramp_v[pl.ds(0, L)] = plsc.cumsum(jnp.ones((L,), i32)) - 1

    dummy16 = jnp.full((L,), DUMMY, i32)

    def _dfill(i, _):
        clist[pl.ds(i * L, L)] = dummy16
        return 0

    lax.fori_loop(0, CLIST // L, _dfill, 0)

    def _bchunk(ch, _):
        ngrp = _stage(ch)

        def _grp(gi, _):
            s16 = src_st[pl.ds(gi * L, L)]
            d16 = dst_st[pl.ds(gi * L, L)]
            rid = d16 >> RSH
            cnt, last = plsc.scan_count(rid)
            base = plsc.load_gather(cur_v, [rid])
            pos = base + cnt - 1
            packed = (s16 << 13) | (d16 & (R - 1))
            plsc.store_scatter(clist, [pos], packed)
            plsc.addupdate_scatter(cur_v, [rid], cnt, mask=last)
            return 0

        return lax.fori_loop(0, ngrp, _grp, 0)

    lax.fori_loop(0, NCHUNK, _bchunk, 0)

    def _zero_range():
        base = s * ROWS_PT
        for t in range(ROWS_PT // 32):
            pltpu.sync_copy(zsrc_v, acc.at[pl.ds(base + t * 32, 32), :])
        rem = ROWS_PT % 32
        if rem:
            pltpu.sync_copy(zsrc_v.at[pl.ds(0, rem), :],
                            acc.at[pl.ds(base + (ROWS_PT // 32) * 32, rem), :])
        for t in range(ROWS_PT // B):
            pltpu.sync_copy(zcnt_v, cnts.at[pl.ds(base + t * B, B), :])
        remc = ROWS_PT % B
        if remc:
            pltpu.sync_copy(zcnt_v.at[pl.ds(0, remc), :],
                            cnts.at[pl.ds(base + (ROWS_PT // B) * B, remc), :])

    def _scatter(start, nb):
        def _batch(bi, _):
            for j in range(B // L):
                packed = clist[pl.ds(start + bi * B + j * L, L)]
                csb_v[pl.ds(j * L, L)] = packed >> 13
                dlb_v[pl.ds(j * L, L)] = packed & 8191
            pltpu.async_copy(wh_hbm.at[csb_v], rows_v, gsem).wait()
            pltpu.sync_copy(rows_v, acc.at[dlb_v], add=True)
            pltpu.sync_copy(ones_v, cnts.at[dlb_v], add=True)
            return 0

        lax.fori_loop(0, nb, _batch, 0)

    def _finalize(lo):
        def _blk(bi, _):
            lrow = (s * (R // L // NS) + bi) * L
            pltpu.sync_copy(acc.at[pl.ds(lrow, L), :], fblk_v)
            pltpu.sync_copy(cnts.at[pl.ds(lrow, L), :], cblk_v)

            def _row(i, _):
                crow = cblk_v[i, pl.ds(0, L)]
                rcpv = 1.0 / jnp.maximum(crow, 1.0)
                for j in range(D // L):
                    fblk_v[i, pl.ds(j * L, L)] = (
                        fblk_v[i, pl.ds(j * L, L)] * rcpv)
                return 0

            lax.fori_loop(0, L, _row, 0)
            grow = lo + lrow

            @pl.when((grow < N) & (c == 0))
            def _():
                pltpu.sync_copy(fblk_v, hb_out.at[pl.ds(grow, L), :])

            @pl.when((grow < N) & (c == 1))
            def _():
                pltpu.sync_copy(fblk_v, ha_out.at[pl.ds(grow, L), :])

            return 0

        lax.fori_loop(0, R // L // NS, _blk, 0)

    for r in range(NR):
        _zero_range()
        plsc.subcore_barrier()
        hv2 = hist_v[pl.ds(0, L)]
        pv2 = poff_v[pl.ds(0, L)]
        _scatter(pv2[r], (hv2[r] + (B - 1)) >> 7)
        plsc.subcore_barrier()
        _finalize(r * R)
        plsc.subcore_barrier()


def _sc_aggregate(wh, src_cat, dst_cat):
    mesh = plsc.VectorSubcoreMesh(core_axis_name="c", subcore_axis_name="s")
    fn = pl.kernel(
        _sc_body,
        out_type=[
            jax.ShapeDtypeStruct((N, D), jnp.float32),
            jax.ShapeDtypeStruct((N, D), jnp.float32),
        ],
        mesh=mesh,
        scratch_types=[
            pltpu.VMEM((CG * L,), jnp.int32),
            pltpu.VMEM((CG * L,), jnp.int32),
            pltpu.VMEM((CLIST,), jnp.int32),
            pltpu.VMEM((B,), jnp.int32),
            pltpu.VMEM((B,), jnp.int32),
            pltpu.VMEM((B, D), jnp.float32),
            pltpu.VMEM((B, L), jnp.float32),
            pltpu.VMEM((32, D), jnp.float32),
            pltpu.VMEM((B, L), jnp.float32),
            pltpu.VMEM((L, D), jnp.float32),
            pltpu.VMEM((L, L), jnp.float32),
            pltpu.VMEM((L,), jnp.int32),
            pltpu.VMEM((L,), jnp.int32),
            pltpu.VMEM((L,), jnp.int32),
            pltpu.VMEM((L,), jnp.int32),
            pltpu.VMEM_SHARED((ACC_ROWS, D), jnp.float32),
            pltpu.VMEM_SHARED((ACC_ROWS, L), jnp.float32),
            pltpu.SemaphoreType.DMA,
        ],
        compiler_params=pltpu.CompilerParams(needs_layout_passes=False),
    )
    return fn(wh, src_cat, dst_cat)


def kernel(x_a, x_b, edge_ab, edge_ba, W_ab, b_ab, W_ba, b_ba):
    x_cat = jnp.concatenate([x_a, x_b], axis=0)
    wt_stack = jnp.stack([W_ab.T, W_ba.T])
    b_stack = jnp.stack([b_ab, b_ba]).reshape(2, 1, D)
    wh = _tc_linear(x_cat, wt_stack, b_stack)

    pad = jnp.zeros((CG * L,), jnp.int32)
    src_cat = jnp.concatenate([
        edge_ab[0].astype(jnp.int32),
        edge_ba[0].astype(jnp.int32) + N,
        pad,
    ])
    dst_cat = jnp.concatenate([
        edge_ab[1].astype(jnp.int32),
        edge_ba[1].astype(jnp.int32),
        pad,
    ])

    use_sc = False
    if use_sc:
        h_b, h_a = _sc_aggregate(wh, src_cat, dst_cat)
    else:
        wh_ab = wh[:N]
        wh_ba = wh[N:]
        sab, dab = edge_ab[0], edge_ab[1]
        sba, dba = edge_ba[0], edge_ba[1]
        ones = jnp.ones((E, 1), jnp.float32)
        sb = jax.ops.segment_sum(jnp.take(wh_ab, sab, axis=0), dab,
                                 num_segments=N)
        cb = jax.ops.segment_sum(ones, dab, num_segments=N)
        h_b = jnp.where(cb > 0, sb / jnp.maximum(cb, 1.0), 0.0)
        sa = jax.ops.segment_sum(jnp.take(wh_ba, sba, axis=0), dba,
                                 num_segments=N)
        ca = jax.ops.segment_sum(ones, dba, num_segments=N)
        h_a = jnp.where(ca > 0, sa / jnp.maximum(ca, 1.0), 0.0)
    return (h_a, h_b)
